# baseline (speedup 1.0000x reference)
import jax
import jax.numpy as jnp
from jax import lax
from jax.experimental import pallas as pl
from jax.experimental.pallas import tpu as pltpu
from jax.experimental.pallas import tpu_sc as plsc

NC = 2
NS = 16
NW = NC * NS

L = 16
NB = 32
_SLICES = (0, 16, 32, 48, 64, 80, 84)


def _sc_body(x_hbm, tab_hbm, out_hbm, xv, idx_v, rows_v, off_v, tab_sh, sem):
    wid = lax.axis_index("s") * NC + lax.axis_index("c")
    n_total, n_genes = x_hbm.shape
    hidden = tab_hbm.shape[1]
    nb_w = n_total // NW
    n_chunks = nb_w // NB

    @pl.when(lax.axis_index("s") == 0)
    def _stage():
        pltpu.sync_copy(tab_hbm, tab_sh)

    iota = lax.iota(jnp.int32, L)
    for st in _SLICES:
        off_v[pl.ds(st, L)] = (iota + st) * 3

    plsc.subcore_barrier()

    def chunk(i, carry):
        nb0 = wid * nb_w + i * NB
        pltpu.sync_copy(x_hbm.at[pl.ds(nb0, NB)], xv)
        for r in range(NB):
            for st in _SLICES:
                sl = pl.ds(st, L)
                idx_v[r, sl] = xv[r, sl] + off_v[sl]
        cps = [
            pltpu.async_copy(
                tab_sh.at[idx_v.at[r]],
                rows_v.at[pl.ds(r * n_genes, n_genes)],
                sem,
            )
            for r in range(NB)
        ]
        for c in cps:
            c.wait()
        pltpu.sync_copy(
            rows_v, out_hbm.at[pl.ds(nb0 * n_genes, NB * n_genes)]
        )
        return carry

    lax.fori_loop(0, n_chunks, chunk, 0)


def kernel(x, emb_tables):
    n, g = x.shape
    _, cat, h = emb_tables.shape
    rows = n * g
    tab = emb_tables.reshape(g * cat, h)

    mesh = plsc.VectorSubcoreMesh(core_axis_name="c", subcore_axis_name="s")
    out = pl.kernel(
        _sc_body,
        out_type=jax.ShapeDtypeStruct((rows, h), jnp.float32),
        mesh=mesh,
        scratch_types=[
            pltpu.VMEM((NB, g), jnp.int32),
            pltpu.VMEM((NB, g), jnp.int32),
            pltpu.VMEM((NB * g, h), jnp.float32),
            pltpu.VMEM((g, ), jnp.int32),
            pltpu.VMEM_SHARED((g * cat, h), jnp.float32),
            pltpu.SemaphoreType.DMA,
        ],
        compiler_params=pltpu.CompilerParams(use_tc_tiling_on_sc=False),
    )(x, tab)
    return out.reshape(n, g, h)


# speedup vs baseline: 1.0029x; 1.0029x over previous
import jax
import jax.numpy as jnp
from jax import lax
from jax.experimental import pallas as pl
from jax.experimental.pallas import tpu as pltpu
from jax.experimental.pallas import tpu_sc as plsc

NB_GENES = 100
CAT_SIZE = 3

NC = 2
NS = 16
NW = NC * NS

L = 16
CHUNK = 3200
XW = 128
PERIOD = 400


def _sc_body(x_hbm, tab_hbm, out_hbm, xv, idx_v, rows_v, off_v, rid_v,
             tab_sh, sem):
    wid = lax.axis_index("s") * NC + lax.axis_index("c")
    b_w = x_hbm.shape[0] * XW // NW
    n_chunks = b_w // CHUNK
    xrows = CHUNK // XW

    @pl.when(lax.axis_index("s") == 0)
    def _stage():
        pltpu.sync_copy(tab_hbm, tab_sh)

    iota = lax.iota(jnp.int32, L)
    for k in range(PERIOD // L):
        off_v[pl.ds(k * L, L)] = ((iota + (k * L)) % NB_GENES) * CAT_SIZE

    plsc.subcore_barrier()

    def chunk(i, carry):
        rbase = wid * (b_w // XW) + i * xrows
        rid_v[pl.ds(0, L)] = rbase + iota
        rid_v[pl.ds(xrows - L, L)] = rbase + (xrows - L) + iota
        pltpu.async_copy(x_hbm.at[rid_v], xv, sem).wait()
        for r in range(xrows):
            for l in range(XW // L):
                ph = (r * XW + l * L) % PERIOD
                sl = pl.ds(l * L, L)
                idx_v[r, sl] = xv[r, sl] + off_v[pl.ds(ph, L)]
        cps = [
            pltpu.async_copy(
                tab_sh.at[idx_v.at[r]],
                rows_v.at[pl.ds(r * XW, XW)],
                sem,
            )
            for r in range(xrows)
        ]
        for c in cps:
            c.wait()
        base = wid * b_w + i * CHUNK
        pltpu.sync_copy(rows_v, out_hbm.at[pl.ds(base, CHUNK)])
        return carry

    lax.fori_loop(0, n_chunks, chunk, 0)


def kernel(x, emb_tables):
    n, g = x.shape
    _, cat, h = emb_tables.shape
    rows = n * g
    x2 = x.reshape(rows // XW, XW)
    tab = emb_tables.reshape(g * cat, h)

    mesh = plsc.VectorSubcoreMesh(core_axis_name="c", subcore_axis_name="s")
    out = pl.kernel(
        _sc_body,
        out_type=jax.ShapeDtypeStruct((rows, h), jnp.float32),
        mesh=mesh,
        scratch_types=[
            pltpu.VMEM((CHUNK // XW, XW), jnp.int32),
            pltpu.VMEM((CHUNK // XW, XW), jnp.int32),
            pltpu.VMEM((CHUNK, h), jnp.float32),
            pltpu.VMEM((PERIOD,), jnp.int32),
            pltpu.VMEM((CHUNK // XW,), jnp.int32),
            pltpu.VMEM_SHARED((g * cat, h), jnp.float32),
            pltpu.SemaphoreType.DMA,
        ],
        compiler_params=pltpu.CompilerParams(use_tc_tiling_on_sc=False),
    )(x2, tab)
    return out.reshape(n, g, h)


# speedup vs baseline: 3.1598x; 3.1506x over previous
import jax
import jax.numpy as jnp
from jax import lax
from jax.experimental import pallas as pl
from jax.experimental.pallas import tpu as pltpu
from jax.experimental.pallas import tpu_sc as plsc

NC = 2
NS = 16
NW = NC * NS

L = 16
NB = 32
XW = 128
_SLICES = (0, 16, 32, 48, 64, 80, 84)


def _sc_body(x_hbm, tab_hbm, out_hbm, xv, idx_v, rows_v, off_v, rid_v,
             tab_sh, sem):
    wid = lax.axis_index("s") * NC + lax.axis_index("c")
    n_total = x_hbm.shape[0]
    n_genes = out_hbm.shape[1]
    nb_w = n_total // NW
    n_chunks = nb_w // NB

    @pl.when(lax.axis_index("s") == 0)
    def _stage():
        pltpu.sync_copy(tab_hbm, tab_sh)

    iota = lax.iota(jnp.int32, L)
    for st in _SLICES:
        off_v[pl.ds(st, L)] = (iota + st) * 3

    plsc.subcore_barrier()

    def chunk(i, carry):
        nb0 = wid * nb_w + i * NB
        rid_v[pl.ds(0, L)] = nb0 + iota
        rid_v[pl.ds(L, L)] = nb0 + L + iota
        pltpu.async_copy(x_hbm.at[rid_v], xv, sem).wait()
        for r in range(NB):
            for st in _SLICES:
                sl = pl.ds(st, L)
                idx_v[r, sl] = xv[r, sl] + off_v[sl]
        cps = [
            pltpu.async_copy(tab_sh.at[idx_v.at[r]], rows_v.at[r], sem)
            for r in range(NB)
        ]
        for c in cps:
            c.wait()
        pltpu.sync_copy(rows_v, out_hbm.at[pl.ds(nb0, NB)])
        return carry

    lax.fori_loop(0, n_chunks, chunk, 0)


def kernel(x, emb_tables):
    n, g = x.shape
    _, cat, h = emb_tables.shape
    xp = jnp.pad(x, ((0, 0), (0, XW - g)))
    tab = emb_tables.reshape(g * cat, h)

    mesh = plsc.VectorSubcoreMesh(core_axis_name="c", subcore_axis_name="s")
    out = pl.kernel(
        _sc_body,
        out_type=jax.ShapeDtypeStruct((n, g, h), jnp.float32),
        mesh=mesh,
        scratch_types=[
            pltpu.VMEM((NB, XW), jnp.int32),
            pltpu.VMEM((NB, g), jnp.int32),
            pltpu.VMEM((NB, g, h), jnp.float32),
            pltpu.VMEM((g,), jnp.int32),
            pltpu.VMEM((NB,), jnp.int32),
            pltpu.VMEM_SHARED((g * cat, h), jnp.float32),
            pltpu.SemaphoreType.DMA,
        ],
        compiler_params=pltpu.CompilerParams(use_tc_tiling_on_sc=False),
    )(xp, tab)
    return out
